# baseline (speedup 1.0000x reference)
import functools

import numpy as np
import jax
import jax.numpy as jnp
from jax import lax
from jax.experimental import pallas as pl
from jax.experimental.pallas import tpu as pltpu
from jax.experimental.pallas import tpu_sc as plsc

_N_LEVELS = 16
_F = 2
_T = 1 << 19
_BASE_RES = 16
_MAX_RES = 2048
_PLS = (_MAX_RES / _BASE_RES) ** (1.0 / (_N_LEVELS - 1))
_N = 524288
_DIM_IN = _N_LEVELS * _F + 3

_NC, _NS, _L = 2, 16, 16
_NW = _NC * _NS
_PW = _N // _NW
_CH = 64
_NCHUNK = _PW // _CH
_NG = _CH // _L
_NCORN = _N_LEVELS * 8

_P2 = np.int32(np.uint32(2654435761).astype(np.int32))
_P3 = np.int32(805459861)
_MASK = np.int32(_T - 1)

_LVL = []
for _l in range(_N_LEVELS):
    _scale_py = _PLS ** _l * _BASE_RES - 1.0
    _res = int(np.ceil(_scale_py)) + 1
    _LVL.append((np.float32(_scale_py), _res, _res ** 3 <= _T))


def _frac_parts(xb0, xb1, xb2, g, scale):
    sl = pl.ds(g * _L, _L)
    out = []
    for xb in (xb0, xb1, xb2):
        pos = (xb[sl] + jnp.float32(0.5)) * scale + jnp.float32(0.5)
        pi = pos.astype(jnp.int32)
        fr = pos - pi.astype(jnp.float32)
        out.append((pi, fr))
    return out


def _encode_body(xx, xy, xz, tab, enc_out, xb0, xb1, xb2, idxb, rowsb, encb,
                 sem):
    wid = lax.axis_index("s") * _NC + lax.axis_index("c")

    def chunk_body(ci, _):
        base = wid * _PW + ci * _CH
        pltpu.sync_copy(xx.at[pl.ds(base, _CH)], xb0)
        pltpu.sync_copy(xy.at[pl.ds(base, _CH)], xb1)
        pltpu.sync_copy(xz.at[pl.ds(base, _CH)], xb2)

        def idx_group(g, _):
            gsl = pl.ds(g * _L, _L)
            for l in range(_N_LEVELS):
                scale, res, dense = _LVL[l]
                (pix, _), (piy, _), (piz, _) = _frac_parts(
                    xb0, xb1, xb2, g, scale)
                lbase = np.int32(l * _T)
                if dense:
                    ax = [pix, pix + np.int32(1)]
                    by0 = piy * np.int32(res)
                    by = [by0, by0 + np.int32(res)]
                    cz0 = piz * np.int32(res * res) + lbase
                    cz = [cz0, cz0 + np.int32(res * res)]
                    for c in range(8):
                        ox, oy, oz = c & 1, (c >> 1) & 1, (c >> 2) & 1
                        gidx = ax[ox] + by[oy] + cz[oz]
                        idxb[pl.ds((l * 8 + c) * _CH + g * _L, _L)] = gidx
                else:
                    hx = [pix, pix + np.int32(1)]
                    hy0 = piy * _P2
                    hy = [hy0, hy0 + _P2]
                    hz0 = piz * _P3
                    hz = [hz0, hz0 + _P3]
                    for c in range(8):
                        ox, oy, oz = c & 1, (c >> 1) & 1, (c >> 2) & 1
                        gidx = ((hx[ox] ^ hy[oy] ^ hz[oz]) & _MASK) + lbase
                        idxb[pl.ds((l * 8 + c) * _CH + g * _L, _L)] = gidx
            return 0

        lax.fori_loop(0, _NG, idx_group, 0)

        pltpu.async_copy(tab.at[idxb], rowsb, sem).wait()

        def interp_group(g, _):
            pids = lax.iota(jnp.int32, _L) + g * _L
            for l in range(_N_LEVELS):
                scale, _, _ = _LVL[l]
                (_, fx), (_, fy), (_, fz) = _frac_parts(
                    xb0, xb1, xb2, g, scale)
                one = jnp.float32(1.0)
                wx = [one - fx, fx]
                wy = [one - fy, fy]
                wz = [one - fz, fz]
                wxy = [wx[0] * wy[0], wx[1] * wy[0], wx[0] * wy[1],
                       wx[1] * wy[1]]
                e0 = jnp.zeros((_L,), jnp.float32)
                e1 = jnp.zeros((_L,), jnp.float32)
                for c in range(8):
                    w = wxy[c & 3] * wz[(c >> 2) & 1]
                    rr = pids + np.int32((l * 8 + c) * _CH)
                    f0 = plsc.load_gather(
                        rowsb, [rr, jnp.zeros((_L,), jnp.int32)])
                    f1 = plsc.load_gather(
                        rowsb, [rr, jnp.ones((_L,), jnp.int32)])
                    e0 = e0 + w * f0
                    e1 = e1 + w * f1
                plsc.store_scatter(
                    encb, [pids, jnp.full((_L,), 2 * l, jnp.int32)], e0)
                plsc.store_scatter(
                    encb, [pids, jnp.full((_L,), 2 * l + 1, jnp.int32)], e1)
            return 0

        lax.fori_loop(0, _NG, interp_group, 0)

        pltpu.sync_copy(encb, enc_out.at[pl.ds(base, _CH)])
        return 0

    lax.fori_loop(0, _NCHUNK, chunk_body, 0)


def _sc_encode(xx, xy, xz, tabf):
    mesh = plsc.VectorSubcoreMesh(core_axis_name="c", subcore_axis_name="s",
                                  num_cores=_NC, num_subcores=_NS)
    f = pl.kernel(
        _encode_body,
        out_type=jax.ShapeDtypeStruct((_N, _N_LEVELS * _F), jnp.float32),
        mesh=mesh,
        scratch_types=[
            pltpu.VMEM((_CH,), jnp.float32),
            pltpu.VMEM((_CH,), jnp.float32),
            pltpu.VMEM((_CH,), jnp.float32),
            pltpu.VMEM((_NCORN * _CH,), jnp.int32),
            pltpu.VMEM((_NCORN * _CH, _F), jnp.float32),
            pltpu.VMEM((_CH, _N_LEVELS * _F), jnp.float32),
            pltpu.SemaphoreType.DMA,
        ],
        compiler_params=pltpu.CompilerParams(needs_layout_passes=False,
                                             use_tc_tiling_on_sc=False),
    )
    return f(xx, xy, xz, tabf)


_BT = 4096


def _mlp_body(x_ref, enc_ref, v1_ref, g1_ref, b1_ref, v2_ref, g2_ref, b2_ref,
              o_ref):
    v1 = v1_ref[...]
    w1 = g1_ref[...] * v1 * lax.rsqrt(
        jnp.sum(v1 * v1, axis=1, keepdims=True))
    hx = lax.dot_general(x_ref[...], w1[:, :3],
                         (((1,), (1,)), ((), ())),
                         preferred_element_type=jnp.float32)
    he = lax.dot_general(enc_ref[...], w1[:, 3:],
                         (((1,), (1,)), ((), ())),
                         preferred_element_type=jnp.float32)
    h = hx + he + b1_ref[...]
    z = h * jnp.float32(100.0)
    sp = jnp.maximum(z, 0.0) + jnp.log1p(jnp.exp(-jnp.abs(z)))
    h2 = sp * jnp.float32(0.01)
    v2 = v2_ref[...]
    w2 = g2_ref[...] * v2 * lax.rsqrt(
        jnp.sum(v2 * v2, axis=1, keepdims=True))
    o_ref[...] = lax.dot_general(h2, w2, (((1,), (1,)), ((), ())),
                                 preferred_element_type=jnp.float32) \
        + b2_ref[...]


def _tc_mlp(x, enc, v1, g1, b1, v2, g2, b2):
    n_out = v2.shape[0]
    grid = (_N // _BT,)
    return pl.pallas_call(
        _mlp_body,
        grid=grid,
        in_specs=[
            pl.BlockSpec((_BT, 3), lambda i: (i, 0)),
            pl.BlockSpec((_BT, _N_LEVELS * _F), lambda i: (i, 0)),
            pl.BlockSpec(v1.shape, lambda i: (0, 0)),
            pl.BlockSpec((v1.shape[0], 1), lambda i: (0, 0)),
            pl.BlockSpec((1, v1.shape[0]), lambda i: (0, 0)),
            pl.BlockSpec(v2.shape, lambda i: (0, 0)),
            pl.BlockSpec((n_out, 1), lambda i: (0, 0)),
            pl.BlockSpec((1, n_out), lambda i: (0, 0)),
        ],
        out_specs=pl.BlockSpec((_BT, n_out), lambda i: (i, 0)),
        out_shape=jax.ShapeDtypeStruct((_N, n_out), jnp.float32),
    )(x, enc, v1, g1.reshape(-1, 1), b1.reshape(1, -1),
      v2, g2.reshape(-1, 1), b2.reshape(1, -1))


def kernel(x, table, v1, g1, b1, v2, g2, b2):
    xt = x.T
    xx, xy, xz = xt[0], xt[1], xt[2]
    tabf = table.reshape(_N_LEVELS * _T, _F)
    enc = _sc_encode(xx, xy, xz, tabf)
    return _tc_mlp(x, enc, v1, g1, b1, v2, g2, b2)


# speedup vs baseline: 3.8713x; 3.8713x over previous
import numpy as np
import jax
import jax.numpy as jnp
from jax import lax
from jax.experimental import pallas as pl
from jax.experimental.pallas import tpu as pltpu
from jax.experimental.pallas import tpu_sc as plsc

_N_LEVELS = 16
_F = 2
_T = 1 << 19
_BASE_RES = 16
_MAX_RES = 2048
_PLS = (_MAX_RES / _BASE_RES) ** (1.0 / (_N_LEVELS - 1))
_N = 524288

_NC, _NS, _L = 2, 16, 16
_NW = _NC * _NS
_PW = _N // _NW
_CH = 128
_NCHUNK = _PW // _CH
_NG = _CH // _L
_NLH = _N_LEVELS // 2
_HC = _NLH * 8 * _CH

_TROWS = _N_LEVELS * _T * _F // 128
_PROWS = _N_LEVELS * _T // 4

_P2 = np.int32(np.uint32(2654435761).astype(np.int32))
_P3 = np.int32(805459861)
_MASK = np.int32(_T - 1)

_LVL = []
for _l in range(_N_LEVELS):
    _scale_py = _PLS ** _l * _BASE_RES - 1.0
    _res = int(np.ceil(_scale_py)) + 1
    _LVL.append((np.float32(_scale_py), _res, _res ** 3 <= _T))

_SC_PARAMS = pltpu.CompilerParams(needs_layout_passes=False,
                                  use_tc_tiling_on_sc=False)


def _mesh():
    return plsc.VectorSubcoreMesh(core_axis_name="c", subcore_axis_name="s",
                                  num_cores=_NC, num_subcores=_NS)


_RC = 64
_RPW = _TROWS // _NW
_RNCH = _RPW // _RC


def _repack_body(tabv, tabp, inb, outb, sem):
    wid = lax.axis_index("s") * _NC + lax.axis_index("c")
    i16 = lax.iota(jnp.int32, _L)
    row_add = lax.shift_right_logical(i16, 2)
    lane_e = (i16 & np.int32(3)) * np.int32(2)

    def chunk(ci, _):
        row0 = wid * _RPW + ci * _RC
        pltpu.sync_copy(tabv.at[pl.ds(row0, _RC)], inb)

        def pair(p, _):
            for g in range(8):
                f0 = inb[2 * p, pl.ds(g * _L, _L)]
                f1 = inb[2 * p + 1, pl.ds(g * _L, _L)]
                rv = p * np.int32(32) + np.int32(g * 4) + row_add
                plsc.store_scatter(outb, [rv, lane_e], f0)
                plsc.store_scatter(outb, [rv, lane_e + np.int32(1)], f1)
            return 0

        lax.fori_loop(0, _RC // 2, pair, 0)
        pltpu.sync_copy(outb, tabp.at[pl.ds((row0 // 2) * 32, 32 * _RC // 2)])
        return 0

    lax.fori_loop(0, _RNCH, chunk, 0)


def _sc_repack(tabv):
    f = pl.kernel(
        _repack_body,
        out_type=jax.ShapeDtypeStruct((_PROWS, 8), jnp.float32),
        mesh=_mesh(),
        scratch_types=[
            pltpu.VMEM((_RC, 128), jnp.float32),
            pltpu.VMEM((32 * _RC // 2, 8), jnp.float32),
            pltpu.SemaphoreType.DMA,
        ],
        compiler_params=_SC_PARAMS,
    )
    return f(tabv)


def _frac_parts(xb0, xb1, xb2, g, scale):
    sl = pl.ds(g * _L, _L)
    out = []
    for xb in (xb0, xb1, xb2):
        pos = (xb[sl] + jnp.float32(0.5)) * scale + jnp.float32(0.5)
        pi = pos.astype(jnp.int32)
        fr = pos - pi.astype(jnp.float32)
        out.append((pi, fr))
    return out


def _encode_body(xx, xy, xz, tabp, enc3, xb0, xb1, xb2, idxb, lowb, rowsb,
                 encb, sem):
    wid = lax.axis_index("s") * _NC + lax.axis_index("c")
    i16 = lax.iota(jnp.int32, _L)

    def chunk_body(ci, _):
        base = wid * _PW + ci * _CH
        pltpu.sync_copy(xx.at[pl.ds(base, _CH)], xb0)
        pltpu.sync_copy(xy.at[pl.ds(base, _CH)], xb1)
        pltpu.sync_copy(xz.at[pl.ds(base, _CH)], xb2)

        for half in range(2):
            lv0 = half * _NLH

            def idx_group(g, _):
                for li in range(_NLH):
                    l = lv0 + li
                    scale, res, dense = _LVL[l]
                    (pix, _), (piy, _), (piz, _) = _frac_parts(
                        xb0, xb1, xb2, g, scale)
                    if dense:
                        ax = [pix, pix + np.int32(1)]
                        by0 = piy * np.int32(res)
                        by = [by0, by0 + np.int32(res)]
                        cz0 = piz * np.int32(res * res)
                        cz = [cz0, cz0 + np.int32(res * res)]
                    else:
                        ax = [pix, pix + np.int32(1)]
                        by0 = piy * _P2
                        by = [by0, by0 + _P2]
                        cz0 = piz * _P3
                        cz = [cz0, cz0 + _P3]
                    for c in range(8):
                        ox, oy, oz = c & 1, (c >> 1) & 1, (c >> 2) & 1
                        if dense:
                            eidx = ax[ox] + by[oy] + cz[oz]
                        else:
                            eidx = (ax[ox] ^ by[oy] ^ cz[oz]) & _MASK
                        off = (li * 8 + c) * _CH + g * _L
                        idxb[pl.ds(off, _L)] = (
                            lax.shift_right_logical(eidx, 2)
                            + np.int32(l * (_T // 4)))
                        lowb[pl.ds(off, _L)] = (eidx & np.int32(3)) \
                            * np.int32(2)
                return 0

            lax.fori_loop(0, _NG, idx_group, 0)

            pltpu.async_copy(tabp.at[idxb], rowsb, sem).wait()

            def interp_group(g, _):
                pids = i16 + g * _L
                for li in range(_NLH):
                    l = lv0 + li
                    scale, _, _ = _LVL[l]
                    (_, fx), (_, fy), (_, fz) = _frac_parts(
                        xb0, xb1, xb2, g, scale)
                    one = jnp.float32(1.0)
                    wx = [one - fx, fx]
                    wy = [one - fy, fy]
                    wz = [one - fz, fz]
                    wxy = [wx[0] * wy[0], wx[1] * wy[0], wx[0] * wy[1],
                           wx[1] * wy[1]]
                    e0 = jnp.zeros((_L,), jnp.float32)
                    e1 = jnp.zeros((_L,), jnp.float32)
                    for c in range(8):
                        w = wxy[c & 3] * wz[(c >> 2) & 1]
                        off = (li * 8 + c) * _CH + g * _L
                        rr = pids + np.int32((li * 8 + c) * _CH)
                        lo = lowb[pl.ds(off, _L)]
                        f0 = plsc.load_gather(rowsb, [rr, lo])
                        f1 = plsc.load_gather(rowsb, [rr, lo + np.int32(1)])
                        e0 = e0 + w * f0
                        e1 = e1 + w * f1
                    for fi, ev in ((0, e0), (1, e1)):
                        fcol = 2 * l + fi
                        enc_off = np.int32((fcol >> 3) * 1024
                                           + (fcol & 7) * 128)
                        plsc.store_scatter(encb, [enc_off + pids], ev)
                return 0

            lax.fori_loop(0, _NG, interp_group, 0)

        blk = wid * _NCHUNK + ci
        for band in range(4):
            pltpu.sync_copy(encb.at[pl.ds(band * 1024, 1024)],
                            enc3.at[band, blk])
        return 0

    lax.fori_loop(0, _NCHUNK, chunk_body, 0)


def _sc_encode(xx, xy, xz, tabp):
    f = pl.kernel(
        _encode_body,
        out_type=jax.ShapeDtypeStruct((4, _N // 128, 1024), jnp.float32),
        mesh=_mesh(),
        scratch_types=[
            pltpu.VMEM((_CH,), jnp.float32),
            pltpu.VMEM((_CH,), jnp.float32),
            pltpu.VMEM((_CH,), jnp.float32),
            pltpu.VMEM((_HC,), jnp.int32),
            pltpu.VMEM((_HC,), jnp.int32),
            pltpu.VMEM((_HC, 8), jnp.float32),
            pltpu.VMEM((4 * 1024,), jnp.float32),
            pltpu.SemaphoreType.DMA,
        ],
        compiler_params=_SC_PARAMS,
    )
    return f(xx, xy, xz, tabp)


_BT = 8192


def _mlp_body(xt_ref, enc_ref, v1_ref, g1_ref, b1_ref, v2_ref, g2_ref,
              b2_ref, o_ref):
    v1 = v1_ref[...]
    w1 = g1_ref[...] * v1 * lax.rsqrt(
        jnp.sum(v1 * v1, axis=1, keepdims=True))
    h = lax.dot_general(w1[:, :3], xt_ref[...], (((1,), (0,)), ((), ())),
                        preferred_element_type=jnp.float32)
    h = h + lax.dot_general(w1[:, 3:], enc_ref[...],
                            (((1,), (0,)), ((), ())),
                            preferred_element_type=jnp.float32)
    h = h + b1_ref[...]
    z = h * jnp.float32(100.0)
    sp = jnp.maximum(z, 0.0) + jnp.log1p(jnp.exp(-jnp.abs(z)))
    h2 = sp * jnp.float32(0.01)
    v2 = v2_ref[...]
    w2 = g2_ref[...] * v2 * lax.rsqrt(
        jnp.sum(v2 * v2, axis=1, keepdims=True))
    o_ref[...] = lax.dot_general(w2, h2, (((1,), (0,)), ((), ())),
                                 preferred_element_type=jnp.float32) \
        + b2_ref[...]


def _tc_mlp(xt, enc_t, v1, g1, b1, v2, g2, b2):
    n_out = v2.shape[0]
    dim_in = v1.shape[1]
    grid = (_N // _BT,)
    return pl.pallas_call(
        _mlp_body,
        grid=grid,
        in_specs=[
            pl.BlockSpec((3, _BT), lambda i: (0, i)),
            pl.BlockSpec((dim_in - 3, _BT), lambda i: (0, i)),
            pl.BlockSpec(v1.shape, lambda i: (0, 0)),
            pl.BlockSpec((v1.shape[0], 1), lambda i: (0, 0)),
            pl.BlockSpec((v1.shape[0], 1), lambda i: (0, 0)),
            pl.BlockSpec(v2.shape, lambda i: (0, 0)),
            pl.BlockSpec((n_out, 1), lambda i: (0, 0)),
            pl.BlockSpec((n_out, 1), lambda i: (0, 0)),
        ],
        out_specs=pl.BlockSpec((n_out, _BT), lambda i: (0, i)),
        out_shape=jax.ShapeDtypeStruct((n_out, _N), jnp.float32),
    )(xt, enc_t, v1, g1.reshape(-1, 1), b1.reshape(-1, 1),
      v2, g2.reshape(-1, 1), b2.reshape(-1, 1))


def kernel(x, table, v1, g1, b1, v2, g2, b2):
    xt = x.T
    xx, xy, xz = xt[0], xt[1], xt[2]
    tabv = table.reshape(_N_LEVELS, _T // 128, 128, _F) \
                .transpose(0, 1, 3, 2).reshape(_TROWS, 128)
    tabp = _sc_repack(tabv)
    enc3 = _sc_encode(xx, xy, xz, tabp)
    enc_t = enc3.reshape(4, _N // 128, 8, 128).transpose(0, 2, 1, 3) \
                .reshape(_N_LEVELS * _F, _N)
    o_t = _tc_mlp(xt, enc_t, v1, g1, b1, v2, g2, b2)
    return o_t.T


# speedup vs baseline: 6.3918x; 1.6511x over previous
import numpy as np
import jax
import jax.numpy as jnp
from jax import lax
from jax.experimental import pallas as pl
from jax.experimental.pallas import tpu as pltpu
from jax.experimental.pallas import tpu_sc as plsc

_N_LEVELS = 16
_F = 2
_T = 1 << 19
_BASE_RES = 16
_MAX_RES = 2048
_PLS = (_MAX_RES / _BASE_RES) ** (1.0 / (_N_LEVELS - 1))
_N = 524288

_NC, _NS, _L = 2, 16, 16
_NW = _NC * _NS
_PW = _N // _NW
_CH = 128
_NCHUNK = _PW // _CH
_NG = _CH // _L
_NLH = _N_LEVELS // 2
_HC = _NLH * 8 * _CH

_TROWS = _N_LEVELS * _T * _F // 128
_PROWS = _N_LEVELS * _T // 4

_P2 = np.int32(np.uint32(2654435761).astype(np.int32))
_P3 = np.int32(805459861)
_MASK = np.int32(_T - 1)

_LVL = []
for _l in range(_N_LEVELS):
    _scale_py = _PLS ** _l * _BASE_RES - 1.0
    _res = int(np.ceil(_scale_py)) + 1
    _LVL.append((np.float32(_scale_py), _res, _res ** 3 <= _T))

_SC_PARAMS = pltpu.CompilerParams(needs_layout_passes=False,
                                  use_tc_tiling_on_sc=False)


def _mesh():
    return plsc.VectorSubcoreMesh(core_axis_name="c", subcore_axis_name="s",
                                  num_cores=_NC, num_subcores=_NS)


_RC = 64
_RPW = _TROWS // _NW
_RNCH = _RPW // _RC


def _repack_body(tabv, tabp, inb, outb, sem):
    wid = lax.axis_index("s") * _NC + lax.axis_index("c")
    i16 = lax.iota(jnp.int32, _L)
    row_add = lax.shift_right_logical(i16, 2)
    lane_e = (i16 & np.int32(3)) * np.int32(2)

    def chunk(ci, _):
        row0 = wid * _RPW + ci * _RC
        pltpu.sync_copy(tabv.at[pl.ds(row0, _RC)], inb)

        def pair(p, _):
            for g in range(8):
                f0 = inb[2 * p, pl.ds(g * _L, _L)]
                f1 = inb[2 * p + 1, pl.ds(g * _L, _L)]
                rv = p * np.int32(32) + np.int32(g * 4) + row_add
                plsc.store_scatter(outb, [rv, lane_e], f0)
                plsc.store_scatter(outb, [rv, lane_e + np.int32(1)], f1)
            return 0

        lax.fori_loop(0, _RC // 2, pair, 0)
        pltpu.sync_copy(outb, tabp.at[pl.ds((row0 // 2) * 32, 32 * _RC // 2)])
        return 0

    lax.fori_loop(0, _RNCH, chunk, 0)


def _sc_repack(tabv):
    f = pl.kernel(
        _repack_body,
        out_type=jax.ShapeDtypeStruct((_PROWS, 8), jnp.float32),
        mesh=_mesh(),
        scratch_types=[
            pltpu.VMEM((_RC, 128), jnp.float32),
            pltpu.VMEM((32 * _RC // 2, 8), jnp.float32),
            pltpu.SemaphoreType.DMA,
        ],
        compiler_params=_SC_PARAMS,
    )
    return f(tabv)


_NLQ = 4
_QC = _NLQ * 8 * _CH


def _corner_parts(pix, piy, piz, l):
    scale, res, dense = _LVL[l]
    if dense:
        ax = [pix, pix + np.int32(1)]
        by0 = piy * np.int32(res)
        by = [by0, by0 + np.int32(res)]
        cz0 = piz * np.int32(res * res)
        cz = [cz0, cz0 + np.int32(res * res)]
    else:
        ax = [pix, pix + np.int32(1)]
        by0 = piy * _P2
        by = [by0, by0 + _P2]
        cz0 = piz * _P3
        cz = [cz0, cz0 + _P3]
    return ax, by, cz, dense


def _encode_body(xx, xy, xz, tabp, enc3, xb0, xb1, xb2, idx0, idx1,
                 rows0, rows1, encb, sem0, sem1, semenc):
    wid = lax.axis_index("s") * _NC + lax.axis_index("c")
    i16 = lax.iota(jnp.int32, _L)
    idxs = (idx0, idx1)
    rows = (rows0, rows1)
    sems = (sem0, sem1)

    def frac_parts(start, scale):
        out = []
        for xb in (xb0, xb1, xb2):
            pos = (xb[pl.ds(start, _L)] + jnp.float32(0.5)) * scale \
                + jnp.float32(0.5)
            pi = pos.astype(jnp.int32)
            fr = pos - pi.astype(jnp.float32)
            out.append((pi, fr))
        return out

    def idx_pass(m, q, k):
        idxb = idxs[k]

        def g_body(g, _):
            start = m * _CH + g * _L
            for li in range(_NLQ):
                l = q * _NLQ + li
                scale, _, _ = _LVL[l]
                (pix, _), (piy, _), (piz, _) = frac_parts(start, scale)
                ax, by, cz, dense = _corner_parts(pix, piy, piz, l)
                for c in range(8):
                    ox, oy, oz = c & 1, (c >> 1) & 1, (c >> 2) & 1
                    if dense:
                        eidx = ax[ox] + by[oy] + cz[oz]
                    else:
                        eidx = (ax[ox] ^ by[oy] ^ cz[oz]) & _MASK
                    off = (li * 8 + c) * _CH + g * _L
                    idxb[pl.ds(off, _L)] = (
                        lax.shift_right_logical(eidx, 2)
                        + np.int32(l * (_T // 4)))
            return 0

        lax.fori_loop(0, _NG, g_body, 0)

    def fire(k):
        pltpu.make_async_copy(tabp.at[idxs[k]], rows[k], sems[k]).start()

    def wait(k):
        pltpu.make_async_copy(tabp.at[idxs[k]], rows[k], sems[k]).wait()

    def interp_pass(m, q, k):
        rowsb = rows[k]

        def g_body(g, _):
            start = m * _CH + g * _L
            pids = i16 + g * _L
            for li in range(_NLQ):
                l = q * _NLQ + li
                scale, res, dense = _LVL[l]
                (pix, fx), (piy, fy), (piz, fz) = frac_parts(start, scale)
                ax, by, cz, _ = _corner_parts(pix, piy, piz, l)
                lx = [(v & np.int32(3)) * np.int32(2) for v in ax]
                ly = [(v & np.int32(3)) * np.int32(2) for v in by]
                lz = [(v & np.int32(3)) * np.int32(2) for v in cz]
                one = jnp.float32(1.0)
                wx = [one - fx, fx]
                wy = [one - fy, fy]
                wz = [one - fz, fz]
                wxy = [wx[0] * wy[0], wx[1] * wy[0], wx[0] * wy[1],
                       wx[1] * wy[1]]
                e0 = jnp.zeros((_L,), jnp.float32)
                e1 = jnp.zeros((_L,), jnp.float32)
                for c in range(8):
                    ox, oy, oz = c & 1, (c >> 1) & 1, (c >> 2) & 1
                    w = wxy[c & 3] * wz[oz]
                    if dense:
                        lo = (lx[ox] + ly[oy] + lz[oz]) & np.int32(6)
                    else:
                        lo = (lx[ox] ^ ly[oy] ^ lz[oz]) & np.int32(6)
                    rr = pids + np.int32((li * 8 + c) * _CH)
                    f0 = plsc.load_gather(rowsb, [rr, lo])
                    f1 = plsc.load_gather(rowsb, [rr, lo + np.int32(1)])
                    e0 = e0 + w * f0
                    e1 = e1 + w * f1
                for fi, ev in ((0, e0), (1, e1)):
                    fcol = 2 * l + fi
                    enc_off = np.int32((fcol >> 3) * 1024 + (fcol & 7) * 128)
                    plsc.store_scatter(encb, [enc_off + pids], ev)
            return 0

        lax.fori_loop(0, _NG, g_body, 0)

    def enc_fire(m):
        blk = wid * _NCHUNK + m
        for band in range(4):
            pltpu.make_async_copy(encb.at[pl.ds(band * 1024, 1024)],
                                  enc3.at[band, blk], semenc).start()

    def enc_wait(m):
        blk = wid * _NCHUNK + m
        for band in range(4):
            pltpu.make_async_copy(encb.at[pl.ds(band * 1024, 1024)],
                                  enc3.at[band, blk], semenc).wait()

    pltpu.sync_copy(xx.at[pl.ds(wid * _PW, _PW)], xb0)
    pltpu.sync_copy(xy.at[pl.ds(wid * _PW, _PW)], xb1)
    pltpu.sync_copy(xz.at[pl.ds(wid * _PW, _PW)], xb2)

    idx_pass(0, 0, 0)
    fire(0)
    idx_pass(0, 1, 1)
    fire(1)

    def chunk_body(m, _):
        @pl.when(m > 0)
        def _():
            enc_wait(m - 1)

        wait(0)
        interp_pass(m, 0, 0)
        idx_pass(m, 2, 0)
        fire(0)
        wait(1)
        interp_pass(m, 1, 1)
        idx_pass(m, 3, 1)
        fire(1)
        wait(0)
        interp_pass(m, 2, 0)

        @pl.when(m < _NCHUNK - 1)
        def _():
            idx_pass(m + 1, 0, 0)
            fire(0)

        wait(1)
        interp_pass(m, 3, 1)

        @pl.when(m < _NCHUNK - 1)
        def _():
            idx_pass(m + 1, 1, 1)
            fire(1)

        enc_fire(m)
        return 0

    lax.fori_loop(0, _NCHUNK, chunk_body, 0)
    enc_wait(_NCHUNK - 1)


def _sc_encode(xx, xy, xz, tabp):
    f = pl.kernel(
        _encode_body,
        out_type=jax.ShapeDtypeStruct((4, _N // 128, 1024), jnp.float32),
        mesh=_mesh(),
        scratch_types=[
            pltpu.VMEM((_PW,), jnp.float32),
            pltpu.VMEM((_PW,), jnp.float32),
            pltpu.VMEM((_PW,), jnp.float32),
            pltpu.VMEM((_QC,), jnp.int32),
            pltpu.VMEM((_QC,), jnp.int32),
            pltpu.VMEM((_QC, 8), jnp.float32),
            pltpu.VMEM((_QC, 8), jnp.float32),
            pltpu.VMEM((4 * 1024,), jnp.float32),
            pltpu.SemaphoreType.DMA,
            pltpu.SemaphoreType.DMA,
            pltpu.SemaphoreType.DMA,
        ],
        compiler_params=_SC_PARAMS,
    )
    return f(xx, xy, xz, tabp)


_BT = 8192


def _mlp_body(xt_ref, enc_ref, v1_ref, g1_ref, b1_ref, v2_ref, g2_ref,
              b2_ref, o_ref):
    v1 = v1_ref[...]
    w1 = g1_ref[...] * v1 * lax.rsqrt(
        jnp.sum(v1 * v1, axis=1, keepdims=True))
    h = lax.dot_general(w1[:, :3], xt_ref[...], (((1,), (0,)), ((), ())),
                        preferred_element_type=jnp.float32)
    h = h + lax.dot_general(w1[:, 3:], enc_ref[...],
                            (((1,), (0,)), ((), ())),
                            preferred_element_type=jnp.float32)
    h = h + b1_ref[...]
    z = h * jnp.float32(100.0)
    sp = jnp.maximum(z, 0.0) + jnp.log1p(jnp.exp(-jnp.abs(z)))
    h2 = sp * jnp.float32(0.01)
    v2 = v2_ref[...]
    w2 = g2_ref[...] * v2 * lax.rsqrt(
        jnp.sum(v2 * v2, axis=1, keepdims=True))
    o_ref[...] = lax.dot_general(w2, h2, (((1,), (0,)), ((), ())),
                                 preferred_element_type=jnp.float32) \
        + b2_ref[...]


def _tc_mlp(xt, enc_t, v1, g1, b1, v2, g2, b2):
    n_out = v2.shape[0]
    dim_in = v1.shape[1]
    grid = (_N // _BT,)
    return pl.pallas_call(
        _mlp_body,
        grid=grid,
        in_specs=[
            pl.BlockSpec((3, _BT), lambda i: (0, i)),
            pl.BlockSpec((dim_in - 3, _BT), lambda i: (0, i)),
            pl.BlockSpec(v1.shape, lambda i: (0, 0)),
            pl.BlockSpec((v1.shape[0], 1), lambda i: (0, 0)),
            pl.BlockSpec((v1.shape[0], 1), lambda i: (0, 0)),
            pl.BlockSpec(v2.shape, lambda i: (0, 0)),
            pl.BlockSpec((n_out, 1), lambda i: (0, 0)),
            pl.BlockSpec((n_out, 1), lambda i: (0, 0)),
        ],
        out_specs=pl.BlockSpec((n_out, _BT), lambda i: (0, i)),
        out_shape=jax.ShapeDtypeStruct((n_out, _N), jnp.float32),
    )(xt, enc_t, v1, g1.reshape(-1, 1), b1.reshape(-1, 1),
      v2, g2.reshape(-1, 1), b2.reshape(-1, 1))


def kernel(x, table, v1, g1, b1, v2, g2, b2):
    xt = x.T
    xx, xy, xz = xt[0], xt[1], xt[2]
    tabv = table.reshape(_N_LEVELS, _T // 128, 128, _F) \
                .transpose(0, 1, 3, 2).reshape(_TROWS, 128)
    tabp = _sc_repack(tabv)
    enc3 = _sc_encode(xx, xy, xz, tabp)
    enc_t = enc3.reshape(4, _N // 128, 8, 128).transpose(0, 2, 1, 3) \
                .reshape(_N_LEVELS * _F, _N)
    o_t = _tc_mlp(xt, enc_t, v1, g1, b1, v2, g2, b2)
    return o_t.T


# speedup vs baseline: 7.8562x; 1.2291x over previous
import numpy as np
import jax
import jax.numpy as jnp
from jax import lax
from jax.experimental import pallas as pl
from jax.experimental.pallas import tpu as pltpu
from jax.experimental.pallas import tpu_sc as plsc

_N_LEVELS = 16
_F = 2
_T = 1 << 19
_BASE_RES = 16
_MAX_RES = 2048
_PLS = (_MAX_RES / _BASE_RES) ** (1.0 / (_N_LEVELS - 1))
_N = 524288

_NC, _NS, _L = 2, 16, 16
_NW = _NC * _NS
_PW = _N // _NW
_CH = 128
_NCHUNK = _PW // _CH
_NG = _CH // _L

_TROWS = _N_LEVELS * _T * _F // 128
_PLAIN_ROWS = _N_LEVELS * _T // 8

_P2 = np.int32(np.uint32(2654435761).astype(np.int32))
_P3 = np.int32(805459861)
_MASK = np.int32(_T - 1)

_LVL = []
for _l in range(_N_LEVELS):
    _scale_py = _PLS ** _l * _BASE_RES - 1.0
    _res = int(np.ceil(_scale_py)) + 1
    _LVL.append((np.float32(_scale_py), _res, _res ** 3 <= _T))

_OCT_AC = 2048
_OBASE = {}
_APW = {}
_rows = _PLAIN_ROWS
for _l in range(_N_LEVELS):
    _, _res, _dense = _LVL[_l]
    if _dense:
        apw = -(-_res ** 3 // _NW)
        _APW[_l] = apw
        _OBASE[_l] = _rows
        _rows += max(_NW * apw, (_NW - 1) * apw + _OCT_AC)
_AROWS = _rows

_SEGS = [[2 * q, 2 * q + 1] for q in range(8)]
_SEG_OFF = []
_SEG_ROWS = []
for _seg in _SEGS:
    offs, tot = [], 0
    for _l in _seg:
        offs.append(tot)
        tot += 1 if _LVL[_l][2] else 8
    _SEG_OFF.append(offs)
    _SEG_ROWS.append(tot)
_MAX_SEG = max(_SEG_ROWS)

_SC_PARAMS = pltpu.CompilerParams(needs_layout_passes=False,
                                  use_tc_tiling_on_sc=False)


def _mesh():
    return plsc.VectorSubcoreMesh(core_axis_name="c", subcore_axis_name="s",
                                  num_cores=_NC, num_subcores=_NS)


_RC = 64
_RPW = _TROWS // _NW
_RNCH = _RPW // _RC


def _repack_body(tabv, tabA, inb, outb, oinb, ooutb, sem):
    wid = lax.axis_index("s") * _NC + lax.axis_index("c")
    i16 = lax.iota(jnp.int32, _L)
    row_add = lax.shift_right_logical(i16, 3)
    lane_e = (i16 & np.int32(7)) * np.int32(2)

    def chunk(ci, _):
        row0 = wid * _RPW + ci * _RC
        pltpu.sync_copy(tabv.at[pl.ds(row0, _RC)], inb)

        def pair(p, _):
            for g in range(8):
                f0 = inb[2 * p, pl.ds(g * _L, _L)]
                f1 = inb[2 * p + 1, pl.ds(g * _L, _L)]
                rv = p * np.int32(16) + np.int32(g * 2) + row_add
                plsc.store_scatter(outb, [rv, lane_e], f0)
                plsc.store_scatter(outb, [rv, lane_e + np.int32(1)], f1)
            return 0

        lax.fori_loop(0, _RC // 2, pair, 0)
        pltpu.sync_copy(outb, tabA.at[pl.ds((row0 // 2) * 16, 16 * _RC // 2)])
        return 0

    lax.fori_loop(0, _RNCH, chunk, 0)

    for l in range(_N_LEVELS):
        _, res, dense = _LVL[l]
        if not dense:
            continue
        apw = _APW[l]
        offs = [ox + oy * res + oz * res * res
                for oz in (0, 1) for oy in (0, 1) for ox in (0, 1)]
        nchunks = -(-apw // _OCT_AC)
        for c in range(nchunks):
            astart = wid * np.int32(apw) \
                + np.int32(min(c * _OCT_AC, max(apw - _OCT_AC, 0)))
            b0 = lax.shift_right_logical(astart, 7)
            delta = astart & np.int32(127)
            pltpu.sync_copy(
                tabv.at[pl.ds((np.int32(l * 4096) + b0) * 2, 92)], oinb)

            def grp(gi, _):
                qloc = gi * _L + i16
                for c8 in range(8):
                    el = delta + gi * _L + i16 + np.int32(offs[c8])
                    blk2 = lax.shift_right_logical(el, 7) * np.int32(2)
                    lane = el & np.int32(127)
                    for f in range(2):
                        v = plsc.load_gather(oinb, [blk2 + np.int32(f), lane])
                        plsc.store_scatter(
                            ooutb,
                            [qloc, jnp.full((_L,), c8 * 2 + f, jnp.int32)], v)
                return 0

            lax.fori_loop(0, _OCT_AC // _L, grp, 0)
            pltpu.sync_copy(ooutb,
                            tabA.at[pl.ds(np.int32(_OBASE[l]) + astart,
                                          _OCT_AC)])


def _sc_repack(tabv):
    f = pl.kernel(
        _repack_body,
        out_type=jax.ShapeDtypeStruct((_AROWS, 16), jnp.float32),
        mesh=_mesh(),
        scratch_types=[
            pltpu.VMEM((_RC, 128), jnp.float32),
            pltpu.VMEM((16 * _RC // 2, 16), jnp.float32),
            pltpu.VMEM((92, 128), jnp.float32),
            pltpu.VMEM((_OCT_AC, 16), jnp.float32),
            pltpu.SemaphoreType.DMA,
        ],
        compiler_params=_SC_PARAMS,
    )
    return f(tabv)


def _encode_body(xx, xy, xz, tabA, enc3, xb0, xb1, xb2, idx0, idx1,
                 rows0, rows1, encb, sem0, sem1, semenc):
    wid = lax.axis_index("s") * _NC + lax.axis_index("c")
    i16 = lax.iota(jnp.int32, _L)
    idxs = (idx0, idx1)
    rows = (rows0, rows1)
    sems = (sem0, sem1)

    def frac_parts(start, scale):
        out = []
        for xb in (xb0, xb1, xb2):
            pos = (xb[pl.ds(start, _L)] + jnp.float32(0.5)) * scale \
                + jnp.float32(0.5)
            pi = pos.astype(jnp.int32)
            fr = pos - pi.astype(jnp.float32)
            out.append((pi, fr))
        return out

    def idx_pass(m, q, k):
        idxb = idxs[k]

        def g_body(g, _):
            start = m * _CH + g * _L
            for li, l in enumerate(_SEGS[q]):
                scale, res, dense = _LVL[l]
                soff = _SEG_OFF[q][li]
                (pix, _), (piy, _), (piz, _) = frac_parts(start, scale)
                if dense:
                    anchor = pix + piy * np.int32(res) \
                        + piz * np.int32(res * res)
                    idxb[pl.ds(soff * _CH + g * _L, _L)] = \
                        anchor + np.int32(_OBASE[l])
                else:
                    hx = [pix, pix + np.int32(1)]
                    hy0 = piy * _P2
                    hy = [hy0, hy0 + _P2]
                    hz0 = piz * _P3
                    hz = [hz0, hz0 + _P3]
                    for c in range(8):
                        ox, oy, oz = c & 1, (c >> 1) & 1, (c >> 2) & 1
                        eidx = (hx[ox] ^ hy[oy] ^ hz[oz]) & _MASK
                        idxb[pl.ds((soff + c) * _CH + g * _L, _L)] = (
                            lax.shift_right_logical(eidx, 3)
                            + np.int32(l * (_T // 8)))
            return 0

        lax.fori_loop(0, _NG, g_body, 0)

    def fire(k, q):
        nr = _SEG_ROWS[q] * _CH
        pltpu.make_async_copy(tabA.at[idxs[k].at[pl.ds(0, nr)]],
                              rows[k].at[pl.ds(0, nr)], sems[k]).start()

    def wait(k, q):
        nr = _SEG_ROWS[q] * _CH
        pltpu.make_async_copy(tabA.at[idxs[k].at[pl.ds(0, nr)]],
                              rows[k].at[pl.ds(0, nr)], sems[k]).wait()

    def interp_pass(m, q, k):
        rowsb = rows[k]

        def g_body(g, _):
            start = m * _CH + g * _L
            pids = i16 + g * _L
            for li, l in enumerate(_SEGS[q]):
                scale, res, dense = _LVL[l]
                soff = _SEG_OFF[q][li]
                (pix, fx), (piy, fy), (piz, fz) = frac_parts(start, scale)
                one = jnp.float32(1.0)
                wx = [one - fx, fx]
                wy = [one - fy, fy]
                wz = [one - fz, fz]
                wxy = [wx[0] * wy[0], wx[1] * wy[0], wx[0] * wy[1],
                       wx[1] * wy[1]]
                e0 = jnp.zeros((_L,), jnp.float32)
                e1 = jnp.zeros((_L,), jnp.float32)
                if dense:
                    rr = pids + np.int32(soff * _CH)
                    for c in range(8):
                        w = wxy[c & 3] * wz[(c >> 2) & 1]
                        f0 = plsc.load_gather(
                            rowsb, [rr, jnp.full((_L,), 2 * c, jnp.int32)])
                        f1 = plsc.load_gather(
                            rowsb, [rr, jnp.full((_L,), 2 * c + 1,
                                                 jnp.int32)])
                        e0 = e0 + w * f0
                        e1 = e1 + w * f1
                else:
                    lx0 = (pix & np.int32(7)) * np.int32(2)
                    lx = [lx0, lx0 + np.int32(2)]
                    ly0 = ((piy * _P2) & np.int32(7)) * np.int32(2)
                    ly = [ly0, ly0 + np.int32((_P2 & 7) * 2)]
                    lz0 = ((piz * _P3) & np.int32(7)) * np.int32(2)
                    lz = [lz0, lz0 + np.int32((_P3 & 7) * 2)]
                    for c in range(8):
                        ox, oy, oz = c & 1, (c >> 1) & 1, (c >> 2) & 1
                        w = wxy[c & 3] * wz[oz]
                        lo = (lx[ox] ^ ly[oy] ^ lz[oz]) & np.int32(14)
                        rr = pids + np.int32((soff + c) * _CH)
                        f0 = plsc.load_gather(rowsb, [rr, lo])
                        f1 = plsc.load_gather(rowsb, [rr, lo + np.int32(1)])
                        e0 = e0 + w * f0
                        e1 = e1 + w * f1
                for fi, ev in ((0, e0), (1, e1)):
                    fcol = 2 * l + fi
                    enc_off = np.int32((fcol >> 3) * 1024 + (fcol & 7) * 128)
                    plsc.store_scatter(encb, [enc_off + pids], ev)
            return 0

        lax.fori_loop(0, _NG, g_body, 0)

    def enc_fire(m):
        blk = wid * _NCHUNK + m
        for band in range(4):
            pltpu.make_async_copy(encb.at[pl.ds(band * 1024, 1024)],
                                  enc3.at[band, blk], semenc).start()

    def enc_wait(m):
        blk = wid * _NCHUNK + m
        for band in range(4):
            pltpu.make_async_copy(encb.at[pl.ds(band * 1024, 1024)],
                                  enc3.at[band, blk], semenc).wait()

    pltpu.sync_copy(xx.at[pl.ds(wid * _PW, _PW)], xb0)
    pltpu.sync_copy(xy.at[pl.ds(wid * _PW, _PW)], xb1)
    pltpu.sync_copy(xz.at[pl.ds(wid * _PW, _PW)], xb2)

    idx_pass(0, 0, 0)
    fire(0, 0)
    idx_pass(0, 1, 1)
    fire(1, 1)

    def chunk_body(m, _):
        @pl.when(m > 0)
        def _():
            enc_wait(m - 1)

        for q in range(8):
            k = q % 2
            wait(k, q)
            interp_pass(m, q, k)
            nq = q + 2
            if nq < 8:
                idx_pass(m, nq, k)
                fire(k, nq)
            else:
                @pl.when(m < _NCHUNK - 1)
                def _(nq=nq, k=k):
                    idx_pass(m + 1, nq - 8, k)
                    fire(k, nq - 8)

        enc_fire(m)
        return 0

    lax.fori_loop(0, _NCHUNK, chunk_body, 0)
    enc_wait(_NCHUNK - 1)


def _sc_encode(xx, xy, xz, tabA):
    f = pl.kernel(
        _encode_body,
        out_type=jax.ShapeDtypeStruct((4, _N // 128, 1024), jnp.float32),
        mesh=_mesh(),
        scratch_types=[
            pltpu.VMEM((_PW,), jnp.float32),
            pltpu.VMEM((_PW,), jnp.float32),
            pltpu.VMEM((_PW,), jnp.float32),
            pltpu.VMEM((_MAX_SEG * _CH,), jnp.int32),
            pltpu.VMEM((_MAX_SEG * _CH,), jnp.int32),
            pltpu.VMEM((_MAX_SEG * _CH, 16), jnp.float32),
            pltpu.VMEM((_MAX_SEG * _CH, 16), jnp.float32),
            pltpu.VMEM((4 * 1024,), jnp.float32),
            pltpu.SemaphoreType.DMA,
            pltpu.SemaphoreType.DMA,
            pltpu.SemaphoreType.DMA,
        ],
        compiler_params=_SC_PARAMS,
    )
    return f(xx, xy, xz, tabA)


_BT = 8192


def _mlp_body(xt_ref, enc_ref, v1_ref, g1_ref, b1_ref, v2_ref, g2_ref,
              b2_ref, o_ref):
    v1 = v1_ref[...]
    w1 = g1_ref[...] * v1 * lax.rsqrt(
        jnp.sum(v1 * v1, axis=1, keepdims=True))
    h = lax.dot_general(w1[:, :3], xt_ref[...], (((1,), (0,)), ((), ())),
                        preferred_element_type=jnp.float32)
    h = h + lax.dot_general(w1[:, 3:], enc_ref[...],
                            (((1,), (0,)), ((), ())),
                            preferred_element_type=jnp.float32)
    h = h + b1_ref[...]
    z = h * jnp.float32(100.0)
    sp = jnp.maximum(z, 0.0) + jnp.log1p(jnp.exp(-jnp.abs(z)))
    h2 = sp * jnp.float32(0.01)
    v2 = v2_ref[...]
    w2 = g2_ref[...] * v2 * lax.rsqrt(
        jnp.sum(v2 * v2, axis=1, keepdims=True))
    o_ref[...] = lax.dot_general(w2, h2, (((1,), (0,)), ((), ())),
                                 preferred_element_type=jnp.float32) \
        + b2_ref[...]


def _tc_mlp(xt, enc_t, v1, g1, b1, v2, g2, b2):
    n_out = v2.shape[0]
    dim_in = v1.shape[1]
    grid = (_N // _BT,)
    return pl.pallas_call(
        _mlp_body,
        grid=grid,
        in_specs=[
            pl.BlockSpec((3, _BT), lambda i: (0, i)),
            pl.BlockSpec((dim_in - 3, _BT), lambda i: (0, i)),
            pl.BlockSpec(v1.shape, lambda i: (0, 0)),
            pl.BlockSpec((v1.shape[0], 1), lambda i: (0, 0)),
            pl.BlockSpec((v1.shape[0], 1), lambda i: (0, 0)),
            pl.BlockSpec(v2.shape, lambda i: (0, 0)),
            pl.BlockSpec((n_out, 1), lambda i: (0, 0)),
            pl.BlockSpec((n_out, 1), lambda i: (0, 0)),
        ],
        out_specs=pl.BlockSpec((n_out, _BT), lambda i: (0, i)),
        out_shape=jax.ShapeDtypeStruct((n_out, _N), jnp.float32),
    )(xt, enc_t, v1, g1.reshape(-1, 1), b1.reshape(-1, 1),
      v2, g2.reshape(-1, 1), b2.reshape(-1, 1))


def kernel(x, table, v1, g1, b1, v2, g2, b2):
    xt = x.T
    xx, xy, xz = xt[0], xt[1], xt[2]
    tabv = table.reshape(_N_LEVELS, _T // 128, 128, _F) \
                .transpose(0, 1, 3, 2).reshape(_TROWS, 128)
    tabA = _sc_repack(tabv)
    enc3 = _sc_encode(xx, xy, xz, tabA)
    enc_t = enc3.reshape(4, _N // 128, 8, 128).transpose(0, 2, 1, 3) \
                .reshape(_N_LEVELS * _F, _N)
    o_t = _tc_mlp(xt, enc_t, v1, g1, b1, v2, g2, b2)
    return o_t.T


# speedup vs baseline: 7.9964x; 1.0179x over previous
import numpy as np
import jax
import jax.numpy as jnp
from jax import lax
from jax.experimental import pallas as pl
from jax.experimental.pallas import tpu as pltpu
from jax.experimental.pallas import tpu_sc as plsc

_N_LEVELS = 16
_F = 2
_T = 1 << 19
_BASE_RES = 16
_MAX_RES = 2048
_PLS = (_MAX_RES / _BASE_RES) ** (1.0 / (_N_LEVELS - 1))
_N = 524288

_NC, _NS, _L = 2, 16, 16
_NW = _NC * _NS
_PW = _N // _NW
_CH = 128
_NCHUNK = _PW // _CH
_NG = _CH // _L

_TROWS = _N_LEVELS * _T * _F // 128
_PLAIN_ROWS = _N_LEVELS * _T // 8

_P2 = np.int32(np.uint32(2654435761).astype(np.int32))
_P3 = np.int32(805459861)
_MASK = np.int32(_T - 1)

_LVL = []
for _l in range(_N_LEVELS):
    _scale_py = _PLS ** _l * _BASE_RES - 1.0
    _res = int(np.ceil(_scale_py)) + 1
    _LVL.append((np.float32(_scale_py), _res, _res ** 3 <= _T))

_OCT_AC = 2048
_OBASE = {}
_APW = {}
_rows = _PLAIN_ROWS
for _l in range(_N_LEVELS):
    _, _res, _dense = _LVL[_l]
    if _dense:
        apw = -(-_res ** 3 // _NW)
        _APW[_l] = apw
        _OBASE[_l] = _rows
        _rows += max(_NW * apw, (_NW - 1) * apw + _OCT_AC)
_AROWS = _rows

_SEGS = [[2 * q, 2 * q + 1] for q in range(8)]
_SEG_OFF = []
_SEG_ROWS = []
for _seg in _SEGS:
    offs, tot = [], 0
    for _l in _seg:
        offs.append(tot)
        tot += 1 if _LVL[_l][2] else 8
    _SEG_OFF.append(offs)
    _SEG_ROWS.append(tot)
_MAX_SEG = max(_SEG_ROWS)

_SC_PARAMS = pltpu.CompilerParams(needs_layout_passes=False,
                                  use_tc_tiling_on_sc=False)


def _mesh():
    return plsc.VectorSubcoreMesh(core_axis_name="c", subcore_axis_name="s",
                                  num_cores=_NC, num_subcores=_NS)


_RC = 256
_RPW = _TROWS // _NW
_RNCH = _RPW // _RC


def _repack_body(tabv, tabA, inb, outb, oinb, ooutb, sem):
    wid = lax.axis_index("s") * _NC + lax.axis_index("c")
    i16 = lax.iota(jnp.int32, _L)
    row_add = lax.shift_right_logical(i16, 3)
    lane_e = (i16 & np.int32(7)) * np.int32(2)

    def chunk(ci, _):
        row0 = wid * _RPW + ci * _RC
        pltpu.sync_copy(tabv.at[pl.ds(row0, _RC)], inb)

        def pair(p, _):
            for g in range(8):
                f0 = inb[2 * p, pl.ds(g * _L, _L)]
                f1 = inb[2 * p + 1, pl.ds(g * _L, _L)]
                rv = p * np.int32(16) + np.int32(g * 2) + row_add
                plsc.store_scatter(outb, [rv, lane_e], f0)
                plsc.store_scatter(outb, [rv, lane_e + np.int32(1)], f1)
            return 0

        lax.fori_loop(0, _RC // 2, pair, 0)
        pltpu.sync_copy(outb, tabA.at[pl.ds((row0 // 2) * 16, 16 * _RC // 2)])
        return 0

    lax.fori_loop(0, _RNCH, chunk, 0)

    for l in range(_N_LEVELS):
        _, res, dense = _LVL[l]
        if not dense:
            continue
        apw = _APW[l]
        offs = [ox + oy * res + oz * res * res
                for oz in (0, 1) for oy in (0, 1) for ox in (0, 1)]
        nchunks = -(-apw // _OCT_AC)
        for c in range(nchunks):
            astart = wid * np.int32(apw) \
                + np.int32(min(c * _OCT_AC, max(apw - _OCT_AC, 0)))
            b0 = lax.shift_right_logical(astart, 7)
            delta = astart & np.int32(127)
            pltpu.sync_copy(
                tabv.at[pl.ds((np.int32(l * 4096) + b0) * 2, 92)], oinb)

            def grp(gi, _):
                qloc = gi * _L + i16
                for c8 in range(8):
                    el = delta + gi * _L + i16 + np.int32(offs[c8])
                    blk2 = lax.shift_right_logical(el, 7) * np.int32(2)
                    lane = el & np.int32(127)
                    for f in range(2):
                        v = plsc.load_gather(oinb, [blk2 + np.int32(f), lane])
                        plsc.store_scatter(
                            ooutb,
                            [qloc, jnp.full((_L,), c8 * 2 + f, jnp.int32)], v)
                return 0

            lax.fori_loop(0, _OCT_AC // _L, grp, 0)
            pltpu.sync_copy(ooutb,
                            tabA.at[pl.ds(np.int32(_OBASE[l]) + astart,
                                          _OCT_AC)])


def _sc_repack(tabv):
    f = pl.kernel(
        _repack_body,
        out_type=jax.ShapeDtypeStruct((_AROWS, 16), jnp.float32),
        mesh=_mesh(),
        scratch_types=[
            pltpu.VMEM((_RC, 128), jnp.float32),
            pltpu.VMEM((16 * _RC // 2, 16), jnp.float32),
            pltpu.VMEM((92, 128), jnp.float32),
            pltpu.VMEM((_OCT_AC, 16), jnp.float32),
            pltpu.SemaphoreType.DMA,
        ],
        compiler_params=_SC_PARAMS,
    )
    return f(tabv)


def _encode_body(xx, xy, xz, tabA, enc3, xb0, xb1, xb2, idx0, idx1,
                 rows0, rows1, encb, sem0, sem1, semenc):
    wid = lax.axis_index("s") * _NC + lax.axis_index("c")
    i16 = lax.iota(jnp.int32, _L)
    idxs = (idx0, idx1)
    rows = (rows0, rows1)
    sems = (sem0, sem1)

    def frac_parts(start, scale):
        out = []
        for xb in (xb0, xb1, xb2):
            pos = (xb[pl.ds(start, _L)] + jnp.float32(0.5)) * scale \
                + jnp.float32(0.5)
            pi = pos.astype(jnp.int32)
            fr = pos - pi.astype(jnp.float32)
            out.append((pi, fr))
        return out

    def idx_pass(m, q, k):
        idxb = idxs[k]

        def g_body(g, _):
            start = m * _CH + g * _L
            for li, l in enumerate(_SEGS[q]):
                scale, res, dense = _LVL[l]
                soff = _SEG_OFF[q][li]
                (pix, _), (piy, _), (piz, _) = frac_parts(start, scale)
                if dense:
                    anchor = pix + piy * np.int32(res) \
                        + piz * np.int32(res * res)
                    idxb[pl.ds(soff * _CH + g * _L, _L)] = \
                        anchor + np.int32(_OBASE[l])
                else:
                    hx = [pix, pix + np.int32(1)]
                    hy0 = piy * _P2
                    hy = [hy0, hy0 + _P2]
                    hz0 = piz * _P3
                    hz = [hz0, hz0 + _P3]
                    for c in range(8):
                        ox, oy, oz = c & 1, (c >> 1) & 1, (c >> 2) & 1
                        eidx = (hx[ox] ^ hy[oy] ^ hz[oz]) & _MASK
                        idxb[pl.ds((soff + c) * _CH + g * _L, _L)] = (
                            lax.shift_right_logical(eidx, 3)
                            + np.int32(l * (_T // 8)))
            return 0

        lax.fori_loop(0, _NG, g_body, 0)

    def fire(k, q):
        nr = _SEG_ROWS[q] * _CH
        pltpu.make_async_copy(tabA.at[idxs[k].at[pl.ds(0, nr)]],
                              rows[k].at[pl.ds(0, nr)], sems[k]).start()

    def wait(k, q):
        nr = _SEG_ROWS[q] * _CH
        pltpu.make_async_copy(tabA.at[idxs[k].at[pl.ds(0, nr)]],
                              rows[k].at[pl.ds(0, nr)], sems[k]).wait()

    def interp_pass(m, q, k):
        rowsb = rows[k]

        def g_body(g, _):
            start = m * _CH + g * _L
            pids = i16 + g * _L
            for li, l in enumerate(_SEGS[q]):
                scale, res, dense = _LVL[l]
                soff = _SEG_OFF[q][li]
                (pix, fx), (piy, fy), (piz, fz) = frac_parts(start, scale)
                one = jnp.float32(1.0)
                wx = [one - fx, fx]
                wy = [one - fy, fy]
                wz = [one - fz, fz]
                wxy = [wx[0] * wy[0], wx[1] * wy[0], wx[0] * wy[1],
                       wx[1] * wy[1]]
                e0 = jnp.zeros((_L,), jnp.float32)
                e1 = jnp.zeros((_L,), jnp.float32)
                if dense:
                    rr = pids + np.int32(soff * _CH)
                    for c in range(8):
                        w = wxy[c & 3] * wz[(c >> 2) & 1]
                        f0 = plsc.load_gather(
                            rowsb, [rr, jnp.full((_L,), 2 * c, jnp.int32)])
                        f1 = plsc.load_gather(
                            rowsb, [rr, jnp.full((_L,), 2 * c + 1,
                                                 jnp.int32)])
                        e0 = e0 + w * f0
                        e1 = e1 + w * f1
                else:
                    lx0 = (pix & np.int32(7)) * np.int32(2)
                    lx = [lx0, lx0 + np.int32(2)]
                    ly0 = ((piy * _P2) & np.int32(7)) * np.int32(2)
                    ly = [ly0, ly0 + np.int32((_P2 & 7) * 2)]
                    lz0 = ((piz * _P3) & np.int32(7)) * np.int32(2)
                    lz = [lz0, lz0 + np.int32((_P3 & 7) * 2)]
                    for c in range(8):
                        ox, oy, oz = c & 1, (c >> 1) & 1, (c >> 2) & 1
                        w = wxy[c & 3] * wz[oz]
                        lo = (lx[ox] ^ ly[oy] ^ lz[oz]) & np.int32(14)
                        rr = pids + np.int32((soff + c) * _CH)
                        f0 = plsc.load_gather(rowsb, [rr, lo])
                        f1 = plsc.load_gather(rowsb, [rr, lo + np.int32(1)])
                        e0 = e0 + w * f0
                        e1 = e1 + w * f1
                for fi, ev in ((0, e0), (1, e1)):
                    fcol = 2 * l + fi
                    enc_off = np.int32((fcol >> 3) * 1024 + (fcol & 7) * 128)
                    plsc.store_scatter(encb, [enc_off + pids], ev)
            return 0

        lax.fori_loop(0, _NG, g_body, 0)

    def enc_fire(m):
        blk = wid * _NCHUNK + m
        for band in range(4):
            pltpu.make_async_copy(encb.at[pl.ds(band * 1024, 1024)],
                                  enc3.at[band, blk], semenc).start()

    def enc_wait(m):
        blk = wid * _NCHUNK + m
        for band in range(4):
            pltpu.make_async_copy(encb.at[pl.ds(band * 1024, 1024)],
                                  enc3.at[band, blk], semenc).wait()

    pltpu.sync_copy(xx.at[pl.ds(wid * _PW, _PW)], xb0)
    pltpu.sync_copy(xy.at[pl.ds(wid * _PW, _PW)], xb1)
    pltpu.sync_copy(xz.at[pl.ds(wid * _PW, _PW)], xb2)

    idx_pass(0, 0, 0)
    fire(0, 0)
    idx_pass(0, 1, 1)
    fire(1, 1)

    def chunk_body(m, _):
        @pl.when(m > 0)
        def _():
            enc_wait(m - 1)

        for q in range(8):
            k = q % 2
            wait(k, q)
            interp_pass(m, q, k)
            nq = q + 2
            if nq < 8:
                idx_pass(m, nq, k)
                fire(k, nq)
            else:
                @pl.when(m < _NCHUNK - 1)
                def _(nq=nq, k=k):
                    idx_pass(m + 1, nq - 8, k)
                    fire(k, nq - 8)

        enc_fire(m)
        return 0

    lax.fori_loop(0, _NCHUNK, chunk_body, 0)
    enc_wait(_NCHUNK - 1)


def _sc_encode(xx, xy, xz, tabA):
    f = pl.kernel(
        _encode_body,
        out_type=jax.ShapeDtypeStruct((4, _N // 128, 1024), jnp.float32),
        mesh=_mesh(),
        scratch_types=[
            pltpu.VMEM((_PW,), jnp.float32),
            pltpu.VMEM((_PW,), jnp.float32),
            pltpu.VMEM((_PW,), jnp.float32),
            pltpu.VMEM((_MAX_SEG * _CH,), jnp.int32),
            pltpu.VMEM((_MAX_SEG * _CH,), jnp.int32),
            pltpu.VMEM((_MAX_SEG * _CH, 16), jnp.float32),
            pltpu.VMEM((_MAX_SEG * _CH, 16), jnp.float32),
            pltpu.VMEM((4 * 1024,), jnp.float32),
            pltpu.SemaphoreType.DMA,
            pltpu.SemaphoreType.DMA,
            pltpu.SemaphoreType.DMA,
        ],
        compiler_params=_SC_PARAMS,
    )
    return f(xx, xy, xz, tabA)


_BT = 8192


def _mlp_body(xt_ref, enc_ref, v1_ref, g1_ref, b1_ref, v2_ref, g2_ref,
              b2_ref, o_ref):
    v1 = v1_ref[...]
    w1 = g1_ref[...] * v1 * lax.rsqrt(
        jnp.sum(v1 * v1, axis=1, keepdims=True))
    h = lax.dot_general(w1[:, :3], xt_ref[...], (((1,), (0,)), ((), ())),
                        preferred_element_type=jnp.float32)
    h = h + lax.dot_general(w1[:, 3:], enc_ref[...],
                            (((1,), (0,)), ((), ())),
                            preferred_element_type=jnp.float32)
    h = h + b1_ref[...]
    z = h * jnp.float32(100.0)
    sp = jnp.maximum(z, 0.0) + jnp.log1p(jnp.exp(-jnp.abs(z)))
    h2 = sp * jnp.float32(0.01)
    v2 = v2_ref[...]
    w2 = g2_ref[...] * v2 * lax.rsqrt(
        jnp.sum(v2 * v2, axis=1, keepdims=True))
    o_ref[...] = lax.dot_general(w2, h2, (((1,), (0,)), ((), ())),
                                 preferred_element_type=jnp.float32) \
        + b2_ref[...]


def _tc_mlp(xt, enc_t, v1, g1, b1, v2, g2, b2):
    n_out = v2.shape[0]
    dim_in = v1.shape[1]
    grid = (_N // _BT,)
    return pl.pallas_call(
        _mlp_body,
        grid=grid,
        in_specs=[
            pl.BlockSpec((3, _BT), lambda i: (0, i)),
            pl.BlockSpec((dim_in - 3, _BT), lambda i: (0, i)),
            pl.BlockSpec(v1.shape, lambda i: (0, 0)),
            pl.BlockSpec((v1.shape[0], 1), lambda i: (0, 0)),
            pl.BlockSpec((v1.shape[0], 1), lambda i: (0, 0)),
            pl.BlockSpec(v2.shape, lambda i: (0, 0)),
            pl.BlockSpec((n_out, 1), lambda i: (0, 0)),
            pl.BlockSpec((n_out, 1), lambda i: (0, 0)),
        ],
        out_specs=pl.BlockSpec((n_out, _BT), lambda i: (0, i)),
        out_shape=jax.ShapeDtypeStruct((n_out, _N), jnp.float32),
    )(xt, enc_t, v1, g1.reshape(-1, 1), b1.reshape(-1, 1),
      v2, g2.reshape(-1, 1), b2.reshape(-1, 1))


def kernel(x, table, v1, g1, b1, v2, g2, b2):
    xt = x.T
    xx, xy, xz = xt[0], xt[1], xt[2]
    tabv = table.reshape(_N_LEVELS, _T // 128, 128, _F) \
                .transpose(0, 1, 3, 2).reshape(_TROWS, 128)
    tabA = _sc_repack(tabv)
    enc3 = _sc_encode(xx, xy, xz, tabA)
    enc_t = enc3.reshape(4, _N // 128, 8, 128).transpose(0, 2, 1, 3) \
                .reshape(_N_LEVELS * _F, _N)
    o_t = _tc_mlp(xt, enc_t, v1, g1, b1, v2, g2, b2)
    return o_t.T


# speedup vs baseline: 8.4273x; 1.0539x over previous
import numpy as np
import jax
import jax.numpy as jnp
from jax import lax
from jax.experimental import pallas as pl
from jax.experimental.pallas import tpu as pltpu
from jax.experimental.pallas import tpu_sc as plsc

_N_LEVELS = 16
_F = 2
_T = 1 << 19
_BASE_RES = 16
_MAX_RES = 2048
_PLS = (_MAX_RES / _BASE_RES) ** (1.0 / (_N_LEVELS - 1))
_N = 524288

_NC, _NS, _L = 2, 16, 16
_NW = _NC * _NS
_PW = _N // _NW
_CH = 128
_NCHUNK = _PW // _CH
_NG = _CH // _L

_TROWS = _N_LEVELS * _T * _F // 128
_PLAIN_ROWS = _N_LEVELS * _T // 8

_P2 = np.int32(np.uint32(2654435761).astype(np.int32))
_P3 = np.int32(805459861)
_MASK = np.int32(_T - 1)

_LVL = []
for _l in range(_N_LEVELS):
    _scale_py = _PLS ** _l * _BASE_RES - 1.0
    _res = int(np.ceil(_scale_py)) + 1
    _LVL.append((np.float32(_scale_py), _res, _res ** 3 <= _T))

_OCT_AC = 2048
_OBASE = {}
_APW = {}
_rows = _PLAIN_ROWS
for _l in range(_N_LEVELS):
    _, _res, _dense = _LVL[_l]
    if _dense:
        apw = -(-_res ** 3 // _NW)
        _APW[_l] = apw
        _OBASE[_l] = _rows
        _rows += max(_NW * apw, (_NW - 1) * apw + _OCT_AC)
_AROWS = _rows

_SEGS = [[q] for q in range(16)]
_NSEG = len(_SEGS)
_NBUF = 4
_SEG_OFF = []
_SEG_ROWS = []
for _seg in _SEGS:
    offs, tot = [], 0
    for _l in _seg:
        offs.append(tot)
        tot += 1 if _LVL[_l][2] else 8
    _SEG_OFF.append(offs)
    _SEG_ROWS.append(tot)
_MAX_SEG = max(_SEG_ROWS)

_SC_PARAMS = pltpu.CompilerParams(needs_layout_passes=False,
                                  use_tc_tiling_on_sc=False)


def _mesh():
    return plsc.VectorSubcoreMesh(core_axis_name="c", subcore_axis_name="s",
                                  num_cores=_NC, num_subcores=_NS)


_RC = 256
_RPW = _TROWS // _NW
_RNCH = _RPW // _RC


def _repack_body(tabv, tabA, inb, outb, oinb, ooutb, sem):
    wid = lax.axis_index("s") * _NC + lax.axis_index("c")
    i16 = lax.iota(jnp.int32, _L)
    row_add = lax.shift_right_logical(i16, 3)
    lane_e = (i16 & np.int32(7)) * np.int32(2)

    def chunk(ci, _):
        row0 = wid * _RPW + ci * _RC
        pltpu.sync_copy(tabv.at[pl.ds(row0, _RC)], inb)

        def pair(p, _):
            for g in range(8):
                f0 = inb[2 * p, pl.ds(g * _L, _L)]
                f1 = inb[2 * p + 1, pl.ds(g * _L, _L)]
                rv = p * np.int32(16) + np.int32(g * 2) + row_add
                plsc.store_scatter(outb, [rv, lane_e], f0)
                plsc.store_scatter(outb, [rv, lane_e + np.int32(1)], f1)
            return 0

        lax.fori_loop(0, _RC // 2, pair, 0)
        pltpu.sync_copy(outb, tabA.at[pl.ds((row0 // 2) * 16, 16 * _RC // 2)])
        return 0

    lax.fori_loop(0, _RNCH, chunk, 0)

    for l in range(_N_LEVELS):
        _, res, dense = _LVL[l]
        if not dense:
            continue
        apw = _APW[l]
        offs = [ox + oy * res + oz * res * res
                for oz in (0, 1) for oy in (0, 1) for ox in (0, 1)]
        nchunks = -(-apw // _OCT_AC)
        for c in range(nchunks):
            astart = wid * np.int32(apw) \
                + np.int32(min(c * _OCT_AC, max(apw - _OCT_AC, 0)))
            b0 = lax.shift_right_logical(astart, 7)
            delta = astart & np.int32(127)
            pltpu.sync_copy(
                tabv.at[pl.ds((np.int32(l * 4096) + b0) * 2, 92)], oinb)

            def grp(gi, _):
                qloc = gi * _L + i16
                for c8 in range(8):
                    el = delta + gi * _L + i16 + np.int32(offs[c8])
                    blk2 = lax.shift_right_logical(el, 7) * np.int32(2)
                    lane = el & np.int32(127)
                    for f in range(2):
                        v = plsc.load_gather(oinb, [blk2 + np.int32(f), lane])
                        plsc.store_scatter(
                            ooutb,
                            [qloc, jnp.full((_L,), c8 * 2 + f, jnp.int32)], v)
                return 0

            lax.fori_loop(0, _OCT_AC // _L, grp, 0)
            pltpu.sync_copy(ooutb,
                            tabA.at[pl.ds(np.int32(_OBASE[l]) + astart,
                                          _OCT_AC)])


def _sc_repack(tabv):
    f = pl.kernel(
        _repack_body,
        out_type=jax.ShapeDtypeStruct((_AROWS, 16), jnp.float32),
        mesh=_mesh(),
        scratch_types=[
            pltpu.VMEM((_RC, 128), jnp.float32),
            pltpu.VMEM((16 * _RC // 2, 16), jnp.float32),
            pltpu.VMEM((92, 128), jnp.float32),
            pltpu.VMEM((_OCT_AC, 16), jnp.float32),
            pltpu.SemaphoreType.DMA,
        ],
        compiler_params=_SC_PARAMS,
    )
    return f(tabv)


def _encode_body(xx, xy, xz, tabA, enc3, xb0, xb1, xb2, idx0, idx1, idx2,
                 idx3, rows0, rows1, rows2, rows3, encb, sem0, sem1, sem2,
                 sem3, semenc):
    wid = lax.axis_index("s") * _NC + lax.axis_index("c")
    i16 = lax.iota(jnp.int32, _L)
    idxs = (idx0, idx1, idx2, idx3)
    rows = (rows0, rows1, rows2, rows3)
    sems = (sem0, sem1, sem2, sem3)

    def frac_parts(start, scale):
        out = []
        for xb in (xb0, xb1, xb2):
            pos = (xb[pl.ds(start, _L)] + jnp.float32(0.5)) * scale \
                + jnp.float32(0.5)
            pi = pos.astype(jnp.int32)
            fr = pos - pi.astype(jnp.float32)
            out.append((pi, fr))
        return out

    def idx_pass(m, q, k):
        idxb = idxs[k]

        def g_body(g, _):
            start = m * _CH + g * _L
            for li, l in enumerate(_SEGS[q]):
                scale, res, dense = _LVL[l]
                soff = _SEG_OFF[q][li]
                (pix, _), (piy, _), (piz, _) = frac_parts(start, scale)
                if dense:
                    anchor = pix + piy * np.int32(res) \
                        + piz * np.int32(res * res)
                    idxb[pl.ds(soff * _CH + g * _L, _L)] = \
                        anchor + np.int32(_OBASE[l])
                else:
                    hx = [pix, pix + np.int32(1)]
                    hy0 = piy * _P2
                    hy = [hy0, hy0 + _P2]
                    hz0 = piz * _P3
                    hz = [hz0, hz0 + _P3]
                    for c in range(8):
                        ox, oy, oz = c & 1, (c >> 1) & 1, (c >> 2) & 1
                        eidx = (hx[ox] ^ hy[oy] ^ hz[oz]) & _MASK
                        idxb[pl.ds((soff + c) * _CH + g * _L, _L)] = (
                            lax.shift_right_logical(eidx, 3)
                            + np.int32(l * (_T // 8)))
            return 0

        lax.fori_loop(0, _NG, g_body, 0)

    def fire(k, q):
        nr = _SEG_ROWS[q] * _CH
        pltpu.make_async_copy(tabA.at[idxs[k].at[pl.ds(0, nr)]],
                              rows[k].at[pl.ds(0, nr)], sems[k]).start()

    def wait(k, q):
        nr = _SEG_ROWS[q] * _CH
        pltpu.make_async_copy(tabA.at[idxs[k].at[pl.ds(0, nr)]],
                              rows[k].at[pl.ds(0, nr)], sems[k]).wait()

    def interp_pass(m, q, k):
        rowsb = rows[k]

        def g_body(g, _):
            start = m * _CH + g * _L
            pids = i16 + g * _L
            for li, l in enumerate(_SEGS[q]):
                scale, res, dense = _LVL[l]
                soff = _SEG_OFF[q][li]
                (pix, fx), (piy, fy), (piz, fz) = frac_parts(start, scale)
                one = jnp.float32(1.0)
                wx = [one - fx, fx]
                wy = [one - fy, fy]
                wz = [one - fz, fz]
                wxy = [wx[0] * wy[0], wx[1] * wy[0], wx[0] * wy[1],
                       wx[1] * wy[1]]
                e0 = jnp.zeros((_L,), jnp.float32)
                e1 = jnp.zeros((_L,), jnp.float32)
                if dense:
                    rr = pids + np.int32(soff * _CH)
                    for c in range(8):
                        w = wxy[c & 3] * wz[(c >> 2) & 1]
                        f0 = plsc.load_gather(
                            rowsb, [rr, jnp.full((_L,), 2 * c, jnp.int32)])
                        f1 = plsc.load_gather(
                            rowsb, [rr, jnp.full((_L,), 2 * c + 1,
                                                 jnp.int32)])
                        e0 = e0 + w * f0
                        e1 = e1 + w * f1
                else:
                    lx0 = (pix & np.int32(7)) * np.int32(2)
                    lx = [lx0, lx0 + np.int32(2)]
                    ly0 = ((piy * _P2) & np.int32(7)) * np.int32(2)
                    ly = [ly0, ly0 + np.int32((_P2 & 7) * 2)]
                    lz0 = ((piz * _P3) & np.int32(7)) * np.int32(2)
                    lz = [lz0, lz0 + np.int32((_P3 & 7) * 2)]
                    for c in range(8):
                        ox, oy, oz = c & 1, (c >> 1) & 1, (c >> 2) & 1
                        w = wxy[c & 3] * wz[oz]
                        lo = (lx[ox] ^ ly[oy] ^ lz[oz]) & np.int32(14)
                        rr = pids + np.int32((soff + c) * _CH)
                        f0 = plsc.load_gather(rowsb, [rr, lo])
                        f1 = plsc.load_gather(rowsb, [rr, lo + np.int32(1)])
                        e0 = e0 + w * f0
                        e1 = e1 + w * f1
                for fi, ev in ((0, e0), (1, e1)):
                    fcol = 2 * l + fi
                    enc_off = np.int32((fcol >> 3) * 1024 + (fcol & 7) * 128)
                    plsc.store_scatter(encb, [enc_off + pids], ev)
            return 0

        lax.fori_loop(0, _NG, g_body, 0)

    def enc_fire(m):
        blk = wid * _NCHUNK + m
        for band in range(4):
            pltpu.make_async_copy(encb.at[pl.ds(band * 1024, 1024)],
                                  enc3.at[band, blk], semenc).start()

    def enc_wait(m):
        blk = wid * _NCHUNK + m
        for band in range(4):
            pltpu.make_async_copy(encb.at[pl.ds(band * 1024, 1024)],
                                  enc3.at[band, blk], semenc).wait()

    pltpu.sync_copy(xx.at[pl.ds(wid * _PW, _PW)], xb0)
    pltpu.sync_copy(xy.at[pl.ds(wid * _PW, _PW)], xb1)
    pltpu.sync_copy(xz.at[pl.ds(wid * _PW, _PW)], xb2)

    for j in range(_NBUF):
        idx_pass(0, j, j)
        fire(j, j)

    def chunk_body(m, _):
        @pl.when(m > 0)
        def _():
            enc_wait(m - 1)

        for q in range(_NSEG):
            k = q % _NBUF
            wait(k, q)
            interp_pass(m, q, k)
            nq = q + _NBUF
            if nq < _NSEG:
                idx_pass(m, nq, k)
                fire(k, nq)
            else:
                @pl.when(m < _NCHUNK - 1)
                def _(nq=nq, k=k):
                    idx_pass(m + 1, nq - _NSEG, k)
                    fire(k, nq - _NSEG)

        enc_fire(m)
        return 0

    lax.fori_loop(0, _NCHUNK, chunk_body, 0)
    enc_wait(_NCHUNK - 1)


def _sc_encode(xx, xy, xz, tabA):
    f = pl.kernel(
        _encode_body,
        out_type=jax.ShapeDtypeStruct((4, _N // 128, 1024), jnp.float32),
        mesh=_mesh(),
        scratch_types=[
            pltpu.VMEM((_PW,), jnp.float32),
            pltpu.VMEM((_PW,), jnp.float32),
            pltpu.VMEM((_PW,), jnp.float32),
            pltpu.VMEM((_MAX_SEG * _CH,), jnp.int32),
            pltpu.VMEM((_MAX_SEG * _CH,), jnp.int32),
            pltpu.VMEM((_MAX_SEG * _CH,), jnp.int32),
            pltpu.VMEM((_MAX_SEG * _CH,), jnp.int32),
            pltpu.VMEM((_MAX_SEG * _CH, 16), jnp.float32),
            pltpu.VMEM((_MAX_SEG * _CH, 16), jnp.float32),
            pltpu.VMEM((_MAX_SEG * _CH, 16), jnp.float32),
            pltpu.VMEM((_MAX_SEG * _CH, 16), jnp.float32),
            pltpu.VMEM((4 * 1024,), jnp.float32),
            pltpu.SemaphoreType.DMA,
            pltpu.SemaphoreType.DMA,
            pltpu.SemaphoreType.DMA,
            pltpu.SemaphoreType.DMA,
            pltpu.SemaphoreType.DMA,
        ],
        compiler_params=_SC_PARAMS,
    )
    return f(xx, xy, xz, tabA)


_BT = 8192


def _mlp_body(xt_ref, enc_ref, v1_ref, g1_ref, b1_ref, v2_ref, g2_ref,
              b2_ref, o_ref):
    v1 = v1_ref[...]
    w1 = g1_ref[...] * v1 * lax.rsqrt(
        jnp.sum(v1 * v1, axis=1, keepdims=True))
    h = lax.dot_general(w1[:, :3], xt_ref[...], (((1,), (0,)), ((), ())),
                        preferred_element_type=jnp.float32)
    h = h + lax.dot_general(w1[:, 3:], enc_ref[...],
                            (((1,), (0,)), ((), ())),
                            preferred_element_type=jnp.float32)
    h = h + b1_ref[...]
    z = h * jnp.float32(100.0)
    sp = jnp.maximum(z, 0.0) + jnp.log1p(jnp.exp(-jnp.abs(z)))
    h2 = sp * jnp.float32(0.01)
    v2 = v2_ref[...]
    w2 = g2_ref[...] * v2 * lax.rsqrt(
        jnp.sum(v2 * v2, axis=1, keepdims=True))
    o_ref[...] = lax.dot_general(w2, h2, (((1,), (0,)), ((), ())),
                                 preferred_element_type=jnp.float32) \
        + b2_ref[...]


def _tc_mlp(xt, enc_t, v1, g1, b1, v2, g2, b2):
    n_out = v2.shape[0]
    dim_in = v1.shape[1]
    grid = (_N // _BT,)
    return pl.pallas_call(
        _mlp_body,
        grid=grid,
        in_specs=[
            pl.BlockSpec((3, _BT), lambda i: (0, i)),
            pl.BlockSpec((dim_in - 3, _BT), lambda i: (0, i)),
            pl.BlockSpec(v1.shape, lambda i: (0, 0)),
            pl.BlockSpec((v1.shape[0], 1), lambda i: (0, 0)),
            pl.BlockSpec((v1.shape[0], 1), lambda i: (0, 0)),
            pl.BlockSpec(v2.shape, lambda i: (0, 0)),
            pl.BlockSpec((n_out, 1), lambda i: (0, 0)),
            pl.BlockSpec((n_out, 1), lambda i: (0, 0)),
        ],
        out_specs=pl.BlockSpec((n_out, _BT), lambda i: (0, i)),
        out_shape=jax.ShapeDtypeStruct((n_out, _N), jnp.float32),
    )(xt, enc_t, v1, g1.reshape(-1, 1), b1.reshape(-1, 1),
      v2, g2.reshape(-1, 1), b2.reshape(-1, 1))


def kernel(x, table, v1, g1, b1, v2, g2, b2):
    xt = x.T
    xx, xy, xz = xt[0], xt[1], xt[2]
    tabv = table.reshape(_N_LEVELS, _T // 128, 128, _F) \
                .transpose(0, 1, 3, 2).reshape(_TROWS, 128)
    tabA = _sc_repack(tabv)
    enc3 = _sc_encode(xx, xy, xz, tabA)
    enc_t = enc3.reshape(4, _N // 128, 8, 128).transpose(0, 2, 1, 3) \
                .reshape(_N_LEVELS * _F, _N)
    o_t = _tc_mlp(xt, enc_t, v1, g1, b1, v2, g2, b2)
    return o_t.T
